# baseline (speedup 1.0000x reference)
import functools

import jax
import jax.numpy as jnp
import numpy as np
from jax import lax
from jax.experimental import pallas as pl
from jax.experimental.pallas import tpu as pltpu
from jax.experimental.pallas import tpu_sc as plsc

N = 4096
B = 16
F = 512
K = 512
N_CHUNK = 1024
LANES = 16
NVEC = N // LANES

MIN_I32 = np.int32(-2147483648)
MSK_I32 = np.int32(2147483647)


def _reduce_body(w_ref, h_ref, o_ref):
    j = pl.program_id(1)
    part = jnp.sum(h_ref[0], axis=1, keepdims=True)

    @pl.when(j == 0)
    def _():
        o_ref[...] = part

    @pl.when(j > 0)
    def _():
        o_ref[...] = o_ref[...] + part

    @pl.when(j == B - 1)
    def _():
        w0 = w_ref[0]
        o_ref[...] = o_ref[...] * (w0 / jnp.sqrt(w0 * w0))


def _scores_tc(H, w):
    return pl.pallas_call(
        _reduce_body,
        grid=(N // N_CHUNK, B),
        in_specs=[
            pl.BlockSpec(memory_space=pltpu.SMEM),
            pl.BlockSpec((1, N_CHUNK, F), lambda i, j: (j, i, 0)),
        ],
        out_specs=pl.BlockSpec((N_CHUNK, 1), lambda i, j: (i, 0)),
        out_shape=jax.ShapeDtypeStruct((N, 1), jnp.float32),
    )(w.reshape(1), H)


def _topk_body(score_hbm, out_hbm, score_v, keys_v, hist_v, out_v, sem):
    wid = lax.axis_index("s") * 2 + lax.axis_index("c")

    @pl.when(wid == 0)
    def _work():
        pltpu.sync_copy(score_hbm, score_v)
        iota = lax.iota(jnp.int32, LANES)
        ones = jnp.ones((LANES,), jnp.int32)

        def _mk(i, _):
            f = score_v[pl.ds(i * LANES, LANES)]
            u = lax.bitcast_convert_type(f, jnp.int32)
            m = lax.shift_right_arithmetic(u, 31)
            kb = u ^ (m & MSK_I32) ^ MIN_I32
            keys_v[pl.ds(i * LANES, LANES)] = kb
            return 0

        lax.fori_loop(0, NVEC, _mk, 0)

        prefix = jnp.int32(0)
        k_rem = jnp.int32(K)
        for p in range(4):
            shift = 24 - 8 * p

            def _zero(i, _):
                hist_v[pl.ds(i * LANES, LANES)] = jnp.zeros((LANES,), jnp.int32)
                return 0

            lax.fori_loop(0, 16, _zero, 0)

            def _scan(i, carry, shift=shift, p=p):
                kb = keys_v[pl.ds(i * LANES, LANES)]
                if p == 0:
                    path = jnp.ones((LANES,), jnp.bool_)
                else:
                    hi = lax.shift_right_logical(kb, shift + 8)
                    path = hi == carry[0]
                bucket = lax.shift_right_logical(kb, shift) & 255
                plsc.addupdate_scatter(hist_v, [bucket], ones, mask=path)
                return carry

            lax.fori_loop(0, NVEC, _scan, (prefix,))

            def _solve(i, carry):
                accum, found, tb, g_above = carry
                j = 15 - i
                hv = hist_v[pl.ds(j * LANES, LANES)]
                rev = lax.rev(hv, (0,))
                c = plsc.cumsum(rev)
                cond = (accum + c) >= k_rem
                npos = jnp.sum(cond.astype(jnp.int32))
                l = plsc.all_reduce_ffs(cond)
                l = jnp.max(l)
                c_l = jnp.sum(jnp.where(iota == l, c, 0))
                h_l = jnp.sum(jnp.where(iota == (15 - l), hv, 0))
                hit = jnp.logical_and(found == 0, npos > 0)
                tb = jnp.where(hit, j * LANES + 15 - l, tb)
                g_above = jnp.where(hit, accum + c_l - h_l, g_above)
                found = jnp.where(hit, jnp.int32(1), found)
                accum = accum + jnp.sum(hv)
                return accum, found, tb, g_above

            _, _, tb, g_above = lax.fori_loop(
                0, 16, _solve,
                (jnp.int32(0), jnp.int32(0), jnp.int32(0), jnp.int32(0)))
            if p == 0:
                prefix = tb
            else:
                prefix = lax.shift_left(prefix, 8) | tb
            k_rem = k_rem - g_above

        t_b = prefix
        need_eq = k_rem

        def _emit(i, carry):
            off, eq_left = carry
            kb = keys_v[pl.ds(i * LANES, LANES)]
            ks = kb ^ MIN_I32
            ts = t_b ^ MIN_I32
            gt = ks > ts
            eq = kb == t_b
            eqc = plsc.cumsum(eq.astype(jnp.int32))
            take_eq = jnp.logical_and(eq, eqc <= eq_left)
            m = jnp.logical_or(gt, take_eq)
            idxv = iota + i * LANES
            plsc.store_compressed(out_v.at[pl.ds(off, LANES)], idxv, mask=m)
            off = off + jnp.sum(m.astype(jnp.int32))
            eq_left = eq_left - jnp.sum(take_eq.astype(jnp.int32))
            return off, eq_left

        lax.fori_loop(0, NVEC, _emit, (jnp.int32(0), need_eq))
        pltpu.sync_copy(out_v.at[pl.ds(0, K)], out_hbm)


@functools.lru_cache(maxsize=None)
def _get_topk_sc():
    return pl.kernel(
        _topk_body,
        out_type=jax.ShapeDtypeStruct((K,), jnp.int32),
        mesh=plsc.VectorSubcoreMesh(core_axis_name="c", subcore_axis_name="s"),
        compiler_params=pltpu.CompilerParams(needs_layout_passes=False),
        scratch_types=[
            pltpu.VMEM((N,), jnp.float32),
            pltpu.VMEM((N,), jnp.int32),
            pltpu.VMEM((256,), jnp.int32),
            pltpu.VMEM((K + LANES,), jnp.int32),
            pltpu.SemaphoreType.DMA,
        ],
    )


def kernel(H, w):
    score = _scores_tc(H, jnp.asarray(w, jnp.float32)).reshape(N)
    return _get_topk_sc()(score)


# speedup vs baseline: 1.3163x; 1.3163x over previous
import functools

import jax
import jax.numpy as jnp
import numpy as np
from jax import lax
from jax.experimental import pallas as pl
from jax.experimental.pallas import tpu as pltpu
from jax.experimental.pallas import tpu_sc as plsc

N = 4096
B = 16
F = 512
K = 512
N_CHUNK = 4096
LANES = 16
NVEC = N // LANES

MIN_I32 = np.int32(-2147483648)
MSK_I32 = np.int32(2147483647)


def _reduce_body(w_ref, h_ref, o_ref):
    j = pl.program_id(1)
    part = jnp.sum(h_ref[0], axis=1, keepdims=True)

    @pl.when(j == 0)
    def _():
        o_ref[...] = part

    @pl.when(j > 0)
    def _():
        o_ref[...] = o_ref[...] + part

    @pl.when(j == B - 1)
    def _():
        w0 = w_ref[0]
        o_ref[...] = o_ref[...] * (w0 / jnp.sqrt(w0 * w0))


def _scores_tc(H, w):
    return pl.pallas_call(
        _reduce_body,
        grid=(N // N_CHUNK, B),
        in_specs=[
            pl.BlockSpec(memory_space=pltpu.SMEM),
            pl.BlockSpec((1, N_CHUNK, F), lambda i, j: (j, i, 0)),
        ],
        out_specs=pl.BlockSpec((N_CHUNK, 1), lambda i, j: (i, 0)),
        out_shape=jax.ShapeDtypeStruct((N, 1), jnp.float32),
    )(w.reshape(1), H)


def _topk_body(score_hbm, out_hbm, score_v, keys_v, hist_v, out_v, sem):
    wid = lax.axis_index("s") * 2 + lax.axis_index("c")

    @pl.when(wid == 0)
    def _work():
        pltpu.sync_copy(score_hbm, score_v)
        iota = lax.iota(jnp.int32, LANES)
        ones = jnp.ones((LANES,), jnp.int32)

        def _mk(i, _):
            f = score_v[pl.ds(i * LANES, LANES)]
            u = lax.bitcast_convert_type(f, jnp.int32)
            m = lax.shift_right_arithmetic(u, 31)
            kb = u ^ (m & MSK_I32) ^ MIN_I32
            keys_v[pl.ds(i * LANES, LANES)] = kb
            return 0

        lax.fori_loop(0, NVEC, _mk, 0)

        prefix = jnp.int32(0)
        k_rem = jnp.int32(K)
        for p in range(4):
            shift = 24 - 8 * p

            def _zero(i, _):
                hist_v[pl.ds(i * LANES, LANES)] = jnp.zeros((LANES,), jnp.int32)
                return 0

            lax.fori_loop(0, 16, _zero, 0)

            def _scan(i, carry, shift=shift, p=p):
                kb = keys_v[pl.ds(i * LANES, LANES)]
                if p == 0:
                    path = jnp.ones((LANES,), jnp.bool_)
                else:
                    hi = lax.shift_right_logical(kb, shift + 8)
                    path = hi == carry[0]
                bucket = lax.shift_right_logical(kb, shift) & 255
                plsc.addupdate_scatter(hist_v, [bucket], ones, mask=path)
                return carry

            lax.fori_loop(0, NVEC, _scan, (prefix,))

            def _solve(i, carry):
                accum, found, tb, g_above = carry
                j = 15 - i
                hv = hist_v[pl.ds(j * LANES, LANES)]
                rev = lax.rev(hv, (0,))
                c = plsc.cumsum(rev)
                cond = (accum + c) >= k_rem
                npos = jnp.sum(cond.astype(jnp.int32))
                l = plsc.all_reduce_ffs(cond)
                l = jnp.max(l)
                c_l = jnp.sum(jnp.where(iota == l, c, 0))
                h_l = jnp.sum(jnp.where(iota == (15 - l), hv, 0))
                hit = jnp.logical_and(found == 0, npos > 0)
                tb = jnp.where(hit, j * LANES + 15 - l, tb)
                g_above = jnp.where(hit, accum + c_l - h_l, g_above)
                found = jnp.where(hit, jnp.int32(1), found)
                accum = accum + jnp.sum(hv)
                return accum, found, tb, g_above

            _, _, tb, g_above = lax.fori_loop(
                0, 16, _solve,
                (jnp.int32(0), jnp.int32(0), jnp.int32(0), jnp.int32(0)))
            if p == 0:
                prefix = tb
            else:
                prefix = lax.shift_left(prefix, 8) | tb
            k_rem = k_rem - g_above

        t_b = prefix
        need_eq = k_rem

        def _emit(i, carry):
            off, eq_left = carry
            kb = keys_v[pl.ds(i * LANES, LANES)]
            ks = kb ^ MIN_I32
            ts = t_b ^ MIN_I32
            gt = ks > ts
            eq = kb == t_b
            eqc = plsc.cumsum(eq.astype(jnp.int32))
            take_eq = jnp.logical_and(eq, eqc <= eq_left)
            m = jnp.logical_or(gt, take_eq)
            idxv = iota + i * LANES
            plsc.store_compressed(out_v.at[pl.ds(off, LANES)], idxv, mask=m)
            off = off + jnp.sum(m.astype(jnp.int32))
            eq_left = eq_left - jnp.sum(take_eq.astype(jnp.int32))
            return off, eq_left

        lax.fori_loop(0, NVEC, _emit, (jnp.int32(0), need_eq))
        pltpu.sync_copy(out_v.at[pl.ds(0, K)], out_hbm)


@functools.lru_cache(maxsize=None)
def _get_topk_sc():
    return pl.kernel(
        _topk_body,
        out_type=jax.ShapeDtypeStruct((K,), jnp.int32),
        mesh=plsc.VectorSubcoreMesh(core_axis_name="c", subcore_axis_name="s"),
        compiler_params=pltpu.CompilerParams(needs_layout_passes=False),
        scratch_types=[
            pltpu.VMEM((N,), jnp.float32),
            pltpu.VMEM((N,), jnp.int32),
            pltpu.VMEM((256,), jnp.int32),
            pltpu.VMEM((K + LANES,), jnp.int32),
            pltpu.SemaphoreType.DMA,
        ],
    )


def kernel(H, w):
    score = _scores_tc(H, jnp.asarray(w, jnp.float32)).reshape(N)
    return _get_topk_sc()(score)
